# baseline (speedup 1.0000x reference)
import functools

import jax
import jax.numpy as jnp
from jax import lax
from jax.experimental import pallas as pl
from jax.experimental.pallas import tpu as pltpu
from jax.experimental.pallas import tpu_sc as plsc

D = 1024
H = 4096
E = 8
T = 2048
K = 2
BT = 128
GR = (T * K + E * (BT - 1) + BT - 1) // BT
R = GR * BT
TB = 256
HC = 128
NH = H // HC
NC, NS = 2, 16
NW = NC * NS
TPW = T // NW
CH = 16
GW = 128


def _router_body(x_ref, gw_ref, gb_ref,
                 g0_ref, g1_ref, e0_ref, e1_ref, r0_ref, r1_ref, cnt_ref,
                 carry):
    i = pl.program_id(0)

    @pl.when(i == 0)
    def _():
        carry[...] = jnp.zeros_like(carry)

    logits = jnp.dot(x_ref[...], gw_ref[...],
                     preferred_element_type=jnp.float32) + gb_ref[...]
    m = jnp.max(logits, axis=1, keepdims=True)
    ex = jnp.exp(logits - m)
    p = ex / jnp.sum(ex, axis=1, keepdims=True)

    ua = lax.broadcasted_iota(jnp.int32, (E, E), 0)
    ub = lax.broadcasted_iota(jnp.int32, (E, E), 1)
    triu = (ua <= ub).astype(jnp.float32)
    lane = lax.broadcasted_iota(jnp.int32, (TB, E), 1).astype(jnp.float32)

    def pick_first_max(q):
        mv = jnp.max(q, axis=1, keepdims=True)
        eq = (q == mv).astype(jnp.float32)
        csum = jnp.dot(eq, triu, preferred_element_type=jnp.float32)
        oh = eq * (csum == 1.0).astype(jnp.float32)
        idx = jnp.sum(oh * lane, axis=1, keepdims=True)
        return mv, oh, idx

    m0, oh0, i0 = pick_first_max(p)
    m1, oh1, i1 = pick_first_max(p - 2.0 * oh0)

    ones16 = jnp.ones((1, GW), jnp.float32)
    g0_ref[...] = m0 * ones16
    g1_ref[...] = m1 * ones16
    e0_ref[...] = i0.astype(jnp.int32)
    e1_ref[...] = i1.astype(jnp.int32)

    ohs = oh0 + oh1
    ra = lax.broadcasted_iota(jnp.int32, (TB, TB), 0)
    rb = lax.broadcasted_iota(jnp.int32, (TB, TB), 1)
    stril = (ra > rb).astype(jnp.float32)
    prior = carry[...] + jnp.dot(stril, ohs, preferred_element_type=jnp.float32)
    r0 = jnp.sum(oh0 * prior, axis=1, keepdims=True)
    r1 = jnp.sum(oh1 * (prior + oh0), axis=1, keepdims=True)
    r0_ref[...] = r0.astype(jnp.int32)
    r1_ref[...] = r1.astype(jnp.int32)

    carry[...] = carry[...] + jnp.sum(ohs, axis=0, keepdims=True)
    cnt_ref[...] = carry[...]


def _router(x2d, gw, gb2d):
    nblk = T // TB
    return pl.pallas_call(
        _router_body,
        grid=(nblk,),
        in_specs=[
            pl.BlockSpec((TB, D), lambda i: (i, 0)),
            pl.BlockSpec((D, E), lambda i: (0, 0)),
            pl.BlockSpec((1, E), lambda i: (0, 0)),
        ],
        out_specs=[
            pl.BlockSpec((TB, GW), lambda i: (i, 0)),
            pl.BlockSpec((TB, GW), lambda i: (i, 0)),
            pl.BlockSpec((TB, 1), lambda i: (i, 0)),
            pl.BlockSpec((TB, 1), lambda i: (i, 0)),
            pl.BlockSpec((TB, 1), lambda i: (i, 0)),
            pl.BlockSpec((TB, 1), lambda i: (i, 0)),
            pl.BlockSpec((1, E), lambda i: (0, 0)),
        ],
        out_shape=[
            jax.ShapeDtypeStruct((T, GW), jnp.float32),
            jax.ShapeDtypeStruct((T, GW), jnp.float32),
            jax.ShapeDtypeStruct((T, 1), jnp.int32),
            jax.ShapeDtypeStruct((T, 1), jnp.int32),
            jax.ShapeDtypeStruct((T, 1), jnp.int32),
            jax.ShapeDtypeStruct((T, 1), jnp.int32),
            jax.ShapeDtypeStruct((1, E), jnp.float32),
        ],
        scratch_shapes=[pltpu.VMEM((1, E), jnp.float32)],
        compiler_params=pltpu.CompilerParams(
            dimension_semantics=("arbitrary",)),
    )(x2d, gw, gb2d)


def _meta_body(cnt_ref, e0_ref, e1_ref, r0_ref, r1_ref,
               pos0_ref, pos1_ref, blk_ref):
    cnt = cnt_ref[...]
    pad = jnp.ceil(cnt / BT) * BT
    ua = lax.broadcasted_iota(jnp.int32, (E, E), 0)
    ub = lax.broadcasted_iota(jnp.int32, (E, E), 1)
    triu = (ua <= ub).astype(jnp.float32)
    upper = jnp.dot(pad, triu, preferred_element_type=jnp.float32)
    off = upper - pad

    lane = lax.broadcasted_iota(jnp.int32, (T, E), 1)

    def to_pos(e_ref, r_ref):
        oh = (lane == e_ref[...]).astype(jnp.float32)
        return (r_ref[...] +
                jnp.sum(oh * off, axis=1, keepdims=True).astype(jnp.int32))

    pos0_ref[...] = to_pos(e0_ref, r0_ref)
    pos1_ref[...] = to_pos(e1_ref, r1_ref)

    brow = lax.broadcasted_iota(jnp.int32, (GR, E), 0).astype(jnp.float32) * BT
    ge = (brow >= upper).astype(jnp.float32)
    blk = jnp.minimum(jnp.sum(ge, axis=1, keepdims=True), float(E - 1))
    blk_ref[...] = blk.astype(jnp.int32)


def _meta(cnt, e0, e1, r0, r1):
    return pl.pallas_call(
        _meta_body,
        out_shape=[
            jax.ShapeDtypeStruct((T, 1), jnp.int32),
            jax.ShapeDtypeStruct((T, 1), jnp.int32),
            jax.ShapeDtypeStruct((GR, 1), jnp.int32),
        ],
    )(cnt, e0, e1, r0, r1)


def _dispatch(x2d, g0s, g1s, pos0w, pos1w):
    mesh = plsc.VectorSubcoreMesh(core_axis_name="c", subcore_axis_name="s",
                                  num_cores=NC, num_subcores=NS)

    @functools.partial(
        pl.kernel,
        out_type=[
            jax.ShapeDtypeStruct((R, D), jnp.float32),
            jax.ShapeDtypeStruct((R, GW), jnp.float32),
        ],
        mesh=mesh,
        scratch_types=[
            pltpu.VMEM((TPW, D), jnp.float32),
            pltpu.VMEM((TPW, GW), jnp.float32),
            pltpu.VMEM((TPW,), jnp.int32),
            pltpu.VMEM((TPW,), jnp.int32),
            pltpu.SemaphoreType.DMA,
        ],
    )
    def k(x_hbm, g0_hbm, g1_hbm, p0_hbm, p1_hbm, xs_hbm, gs_hbm,
          xbuf, gbuf, idx0, idx1, sem):
        wid = lax.axis_index("s") * NC + lax.axis_index("c")
        base = wid * TPW
        pltpu.sync_copy(p0_hbm.at[wid], idx0)
        pltpu.sync_copy(p1_hbm.at[wid], idx1)
        pltpu.sync_copy(x_hbm.at[pl.ds(base, TPW)], xbuf)
        pltpu.async_copy(xbuf, xs_hbm.at[idx0], sem).wait()
        pltpu.async_copy(xbuf, xs_hbm.at[idx1], sem).wait()
        pltpu.sync_copy(g0_hbm.at[pl.ds(base, TPW)], gbuf)
        pltpu.async_copy(gbuf, gs_hbm.at[idx0], sem).wait()
        pltpu.sync_copy(g1_hbm.at[pl.ds(base, TPW)], gbuf)
        pltpu.async_copy(gbuf, gs_hbm.at[idx1], sem).wait()

    return k(x2d, g0s, g1s, pos0w, pos1w)


def _cast_bf16_body(x_ref, o_ref):
    o_ref[...] = x_ref[...].astype(jnp.bfloat16)


def _cast_bf16(x2d):
    n, d = x2d.shape
    return pl.pallas_call(
        _cast_bf16_body,
        grid=(n // 512,),
        in_specs=[pl.BlockSpec((512, d), lambda i: (i, 0))],
        out_specs=pl.BlockSpec((512, d), lambda i: (i, 0)),
        out_shape=jax.ShapeDtypeStruct((n, d), jnp.bfloat16),
    )(x2d)


def _gffn_body(blk_ref, kb_ref, kbias_ref, vb_ref, vbias_ref, xsb_ref, gs_ref,
               out_ref, kbf, vbf):
    hc = pl.program_id(0)
    for e in range(E):
        kbf[pl.ds(e * D, D), :] = kb_ref[:, e, :].astype(jnp.bfloat16)
        vbf[pl.ds(e * HC, HC), :] = vb_ref[:, e, :].astype(jnp.bfloat16)

    def b_body(b, _):
        e = blk_ref[b]
        sl = pl.ds(b * BT, BT)
        xb = xsb_ref[sl, :]
        kp = kbf[pl.ds(e * D, D), :]
        h = jnp.dot(xb, kp, preferred_element_type=jnp.float32)
        brow = kbias_ref[0:1, :]
        vrow = vbias_ref[0:1, :]
        for ee in range(1, E):
            brow = jnp.where(e == ee, kbias_ref[ee:ee + 1, :], brow)
            vrow = jnp.where(e == ee, vbias_ref[ee:ee + 1, :], vrow)
        h = jax.nn.gelu(h + brow)
        vp = vbf[pl.ds(e * HC, HC), :]
        y = jnp.dot(h.astype(jnp.bfloat16), vp,
                    preferred_element_type=jnp.float32)
        prev = jnp.where(hc == 0, vrow, out_ref[sl, :])
        new = prev + y
        new = jnp.where(hc == NH - 1, new * gs_ref[sl, 0:1], new)
        out_ref[sl, :] = new
        return 0

    lax.fori_loop(0, GR, b_body, 0)


def _gffn(blk, keys, key_bias, values, value_bias, xsb, gs):
    spec = pltpu.PrefetchScalarGridSpec(
        num_scalar_prefetch=1,
        grid=(NH,),
        in_specs=[
            pl.BlockSpec((D, E, HC), lambda i, b: (0, 0, i)),
            pl.BlockSpec((E, HC), lambda i, b: (0, i)),
            pl.BlockSpec((HC, E, D), lambda i, b: (i, 0, 0)),
            pl.BlockSpec((E, D), lambda i, b: (0, 0)),
            pl.BlockSpec((R, D), lambda i, b: (0, 0)),
            pl.BlockSpec((R, GW), lambda i, b: (0, 0)),
        ],
        out_specs=pl.BlockSpec((R, D), lambda i, b: (0, 0)),
        scratch_shapes=[pltpu.VMEM((E * D, HC), jnp.bfloat16),
                        pltpu.VMEM((E * HC, D), jnp.bfloat16)],
    )
    return pl.pallas_call(
        _gffn_body,
        grid_spec=spec,
        out_shape=jax.ShapeDtypeStruct((R, D), jnp.float32),
        compiler_params=pltpu.CompilerParams(
            dimension_semantics=("arbitrary",)),
    )(blk, keys, key_bias, values, value_bias, xsb, gs)


def _sffn_body(sk_ref, skb_ref, sv_ref, svb_ref, xb_ref, o_ref, skf, svf):
    hc = pl.program_id(0)
    skf[...] = sk_ref[:, 0, :].astype(jnp.bfloat16)
    svf[...] = sv_ref[:, 0, :].astype(jnp.bfloat16)

    def b_body(b, _):
        sl = pl.ds(b * BT, BT)
        h = jnp.dot(xb_ref[sl, :], skf[...], preferred_element_type=jnp.float32)
        h = jax.nn.gelu(h + skb_ref[0:1, :])
        y = jnp.dot(h.astype(jnp.bfloat16), svf[...],
                    preferred_element_type=jnp.float32)
        prev = jnp.where(hc == 0, svb_ref[0:1, :], o_ref[sl, :])
        o_ref[sl, :] = prev + y
        return 0

    lax.fori_loop(0, T // BT, b_body, 0)


def _sffn(sk, skb, sv, svb, xb):
    return pl.pallas_call(
        _sffn_body,
        grid=(NH,),
        in_specs=[
            pl.BlockSpec((D, 1, HC), lambda i: (0, 0, i)),
            pl.BlockSpec((1, HC), lambda i: (0, i)),
            pl.BlockSpec((HC, 1, D), lambda i: (i, 0, 0)),
            pl.BlockSpec((1, D), lambda i: (0, 0)),
            pl.BlockSpec((T, D), lambda i: (0, 0)),
        ],
        out_specs=pl.BlockSpec((T, D), lambda i: (0, 0)),
        out_shape=jax.ShapeDtypeStruct((T, D), jnp.float32),
        scratch_shapes=[pltpu.VMEM((D, HC), jnp.bfloat16),
                        pltpu.VMEM((HC, D), jnp.bfloat16)],
        compiler_params=pltpu.CompilerParams(
            dimension_semantics=("arbitrary",)),
    )(sk, skb, sv, svb, xb)


def _combine(ys, sh, pos0w, pos1w):
    mesh = plsc.VectorSubcoreMesh(core_axis_name="c", subcore_axis_name="s",
                                  num_cores=NC, num_subcores=NS)

    @functools.partial(
        pl.kernel,
        out_type=jax.ShapeDtypeStruct((T, D), jnp.float32),
        mesh=mesh,
        scratch_types=[
            pltpu.VMEM((CH, D), jnp.float32),
            pltpu.VMEM((CH, D), jnp.float32),
            pltpu.VMEM((CH, D), jnp.float32),
            pltpu.VMEM((TPW,), jnp.int32),
            pltpu.VMEM((TPW,), jnp.int32),
            pltpu.SemaphoreType.DMA,
        ],
    )
    def k(ys_hbm, sh_hbm, p0_hbm, p1_hbm, out_hbm,
          shbuf, b0, b1, idx0, idx1, sem):
        wid = lax.axis_index("s") * NC + lax.axis_index("c")
        base = wid * TPW
        pltpu.sync_copy(p0_hbm.at[wid], idx0)
        pltpu.sync_copy(p1_hbm.at[wid], idx1)
        for j in range(TPW // CH):
            off = j * CH
            iv0 = idx0[pl.ds(off, CH)]
            iv1 = idx1[pl.ds(off, CH)]
            pltpu.sync_copy(sh_hbm.at[pl.ds(base + off, CH)], shbuf)
            pltpu.async_copy(ys_hbm.at[iv0], b0, sem).wait()
            pltpu.async_copy(ys_hbm.at[iv1], b1, sem).wait()

            def row_body(r, _):
                def col_body(c, _):
                    sl = pl.ds(c * 16, 16)
                    shbuf[r, sl] = shbuf[r, sl] + b0[r, sl] + b1[r, sl]
                    return 0
                lax.fori_loop(0, D // 16, col_body, 0)
                return 0

            lax.fori_loop(0, CH, row_body, 0)
            pltpu.sync_copy(shbuf, out_hbm.at[pl.ds(base + off, CH)])

    return k(ys, sh, pos0w, pos1w)


def kernel(x, gate_weight, gate_bias, keys, key_bias, values, value_bias,
           s_keys, s_key_bias, s_values, s_value_bias):
    x2d = x.reshape(T, D)
    gb2d = gate_bias.reshape(1, E)
    skb = s_key_bias.reshape(1, H)
    svb = s_value_bias.reshape(1, D)

    g0s, g1s, e0, e1, r0, r1, cnt = _router(x2d, gate_weight, gb2d)
    pos0, pos1, blk = _meta(cnt, e0, e1, r0, r1)
    pos0w = pos0.reshape(NW, TPW)
    pos1w = pos1.reshape(NW, TPW)

    xs, gs = _dispatch(x2d, g0s, g1s, pos0w, pos1w)
    xsb = _cast_bf16(xs)
    ys = _gffn(blk.reshape(GR), keys, key_bias, values, value_bias, xsb, gs)
    sh = _sffn(s_keys, skb, s_values, svb, _cast_bf16(x2d))
    out = _combine(ys, sh, pos0w, pos1w)
    return out.reshape(x.shape)


# speedup vs baseline: 2.3517x; 2.3517x over previous
import functools

import jax
import jax.numpy as jnp
from jax import lax
from jax.experimental import pallas as pl
from jax.experimental.pallas import tpu as pltpu
from jax.experimental.pallas import tpu_sc as plsc

D = 1024
H = 4096
E = 8
T = 2048
K = 2
BT = 128
GR = (T * K + E * (BT - 1) + BT - 1) // BT
R = GR * BT
TB = 256
HC = 128
NH = H // HC
NC, NS = 2, 16
NW = NC * NS
TPW = T // NW
CH = 16
GW = 128


def _router_body(x_ref, gw_ref, gb_ref,
                 g0_ref, g1_ref, e0_ref, e1_ref, r0_ref, r1_ref, cnt_ref,
                 carry):
    i = pl.program_id(0)

    @pl.when(i == 0)
    def _():
        carry[...] = jnp.zeros_like(carry)

    logits = jnp.dot(x_ref[...], gw_ref[...],
                     preferred_element_type=jnp.float32) + gb_ref[...]
    m = jnp.max(logits, axis=1, keepdims=True)
    ex = jnp.exp(logits - m)
    p = ex / jnp.sum(ex, axis=1, keepdims=True)

    ua = lax.broadcasted_iota(jnp.int32, (E, E), 0)
    ub = lax.broadcasted_iota(jnp.int32, (E, E), 1)
    triu = (ua <= ub).astype(jnp.float32)
    lane = lax.broadcasted_iota(jnp.int32, (TB, E), 1).astype(jnp.float32)

    def pick_first_max(q):
        mv = jnp.max(q, axis=1, keepdims=True)
        eq = (q == mv).astype(jnp.float32)
        csum = jnp.dot(eq, triu, preferred_element_type=jnp.float32)
        oh = eq * (csum == 1.0).astype(jnp.float32)
        idx = jnp.sum(oh * lane, axis=1, keepdims=True)
        return mv, oh, idx

    m0, oh0, i0 = pick_first_max(p)
    m1, oh1, i1 = pick_first_max(p - 2.0 * oh0)

    ones16 = jnp.ones((1, GW), jnp.float32)
    g0_ref[...] = m0 * ones16
    g1_ref[...] = m1 * ones16
    e0_ref[...] = i0.astype(jnp.int32)
    e1_ref[...] = i1.astype(jnp.int32)

    ohs = oh0 + oh1
    ra = lax.broadcasted_iota(jnp.int32, (TB, TB), 0)
    rb = lax.broadcasted_iota(jnp.int32, (TB, TB), 1)
    stril = (ra > rb).astype(jnp.float32)
    prior = carry[...] + jnp.dot(stril, ohs, preferred_element_type=jnp.float32)
    r0 = jnp.sum(oh0 * prior, axis=1, keepdims=True)
    r1 = jnp.sum(oh1 * (prior + oh0), axis=1, keepdims=True)
    r0_ref[...] = r0.astype(jnp.int32)
    r1_ref[...] = r1.astype(jnp.int32)

    carry[...] = carry[...] + jnp.sum(ohs, axis=0, keepdims=True)
    cnt_ref[...] = carry[...]


def _router(x2d, gw, gb2d):
    nblk = T // TB
    return pl.pallas_call(
        _router_body,
        grid=(nblk,),
        in_specs=[
            pl.BlockSpec((TB, D), lambda i: (i, 0)),
            pl.BlockSpec((D, E), lambda i: (0, 0)),
            pl.BlockSpec((1, E), lambda i: (0, 0)),
        ],
        out_specs=[
            pl.BlockSpec((TB, GW), lambda i: (i, 0)),
            pl.BlockSpec((TB, GW), lambda i: (i, 0)),
            pl.BlockSpec((TB, 1), lambda i: (i, 0)),
            pl.BlockSpec((TB, 1), lambda i: (i, 0)),
            pl.BlockSpec((TB, 1), lambda i: (i, 0)),
            pl.BlockSpec((TB, 1), lambda i: (i, 0)),
            pl.BlockSpec((1, E), lambda i: (0, 0)),
        ],
        out_shape=[
            jax.ShapeDtypeStruct((T, GW), jnp.float32),
            jax.ShapeDtypeStruct((T, GW), jnp.float32),
            jax.ShapeDtypeStruct((T, 1), jnp.int32),
            jax.ShapeDtypeStruct((T, 1), jnp.int32),
            jax.ShapeDtypeStruct((T, 1), jnp.int32),
            jax.ShapeDtypeStruct((T, 1), jnp.int32),
            jax.ShapeDtypeStruct((1, E), jnp.float32),
        ],
        scratch_shapes=[pltpu.VMEM((1, E), jnp.float32)],
        compiler_params=pltpu.CompilerParams(
            dimension_semantics=("arbitrary",)),
    )(x2d, gw, gb2d)


def _meta_body(cnt_ref, e0_ref, e1_ref, r0_ref, r1_ref,
               pos0_ref, pos1_ref, blk_ref):
    cnt = cnt_ref[...]
    pad = jnp.ceil(cnt / BT) * BT
    ua = lax.broadcasted_iota(jnp.int32, (E, E), 0)
    ub = lax.broadcasted_iota(jnp.int32, (E, E), 1)
    triu = (ua <= ub).astype(jnp.float32)
    upper = jnp.dot(pad, triu, preferred_element_type=jnp.float32)
    off = upper - pad

    lane = lax.broadcasted_iota(jnp.int32, (T, E), 1)

    def to_pos(e_ref, r_ref):
        oh = (lane == e_ref[...]).astype(jnp.float32)
        return (r_ref[...] +
                jnp.sum(oh * off, axis=1, keepdims=True).astype(jnp.int32))

    pos0_ref[...] = to_pos(e0_ref, r0_ref)
    pos1_ref[...] = to_pos(e1_ref, r1_ref)

    brow = lax.broadcasted_iota(jnp.int32, (GR, E), 0).astype(jnp.float32) * BT
    ge = (brow >= upper).astype(jnp.float32)
    blk = jnp.minimum(jnp.sum(ge, axis=1, keepdims=True), float(E - 1))
    blk_ref[...] = blk.astype(jnp.int32)


def _meta(cnt, e0, e1, r0, r1):
    return pl.pallas_call(
        _meta_body,
        out_shape=[
            jax.ShapeDtypeStruct((T, 1), jnp.int32),
            jax.ShapeDtypeStruct((T, 1), jnp.int32),
            jax.ShapeDtypeStruct((GR, 1), jnp.int32),
        ],
    )(cnt, e0, e1, r0, r1)


def _dispatch(x2d, g0s, g1s, pos0w, pos1w):
    mesh = plsc.VectorSubcoreMesh(core_axis_name="c", subcore_axis_name="s",
                                  num_cores=NC, num_subcores=NS)

    @functools.partial(
        pl.kernel,
        out_type=[
            jax.ShapeDtypeStruct((R, D), jnp.float32),
            jax.ShapeDtypeStruct((R, GW), jnp.float32),
        ],
        mesh=mesh,
        scratch_types=[
            pltpu.VMEM((TPW, D), jnp.float32),
            pltpu.VMEM((TPW, GW), jnp.float32),
            pltpu.VMEM((TPW,), jnp.int32),
            pltpu.VMEM((TPW,), jnp.int32),
            pltpu.SemaphoreType.DMA,
        ],
    )
    def k(x_hbm, g0_hbm, g1_hbm, p0_hbm, p1_hbm, xs_hbm, gs_hbm,
          xbuf, gbuf, idx0, idx1, sem):
        wid = lax.axis_index("s") * NC + lax.axis_index("c")
        base = wid * TPW
        pltpu.sync_copy(p0_hbm.at[wid], idx0)
        pltpu.sync_copy(p1_hbm.at[wid], idx1)
        pltpu.sync_copy(x_hbm.at[pl.ds(base, TPW)], xbuf)
        pltpu.async_copy(xbuf, xs_hbm.at[idx0], sem).wait()
        pltpu.async_copy(xbuf, xs_hbm.at[idx1], sem).wait()
        pltpu.sync_copy(g0_hbm.at[pl.ds(base, TPW)], gbuf)
        pltpu.async_copy(gbuf, gs_hbm.at[idx0], sem).wait()
        pltpu.sync_copy(g1_hbm.at[pl.ds(base, TPW)], gbuf)
        pltpu.async_copy(gbuf, gs_hbm.at[idx1], sem).wait()

    return k(x2d, g0s, g1s, pos0w, pos1w)


def _cast_bf16_body(x_ref, o_ref):
    o_ref[...] = x_ref[...].astype(jnp.bfloat16)


def _cast_bf16(x2d):
    n, d = x2d.shape
    return pl.pallas_call(
        _cast_bf16_body,
        grid=(n // 512,),
        in_specs=[pl.BlockSpec((512, d), lambda i: (i, 0))],
        out_specs=pl.BlockSpec((512, d), lambda i: (i, 0)),
        out_shape=jax.ShapeDtypeStruct((n, d), jnp.bfloat16),
    )(x2d)


def _gffn_body(blk_ref, xs_ref, gs_ref, kb_ref, kbias_ref, vb_ref, vbias_ref,
               ys_ref):
    xb = xs_ref[...].astype(jnp.bfloat16)
    h = jnp.dot(xb, kb_ref[...], preferred_element_type=jnp.float32)
    h = jax.nn.gelu(h + kbias_ref[...])
    y = jnp.dot(h.astype(jnp.bfloat16), vb_ref[...],
                preferred_element_type=jnp.float32)
    ys_ref[...] = (y + vbias_ref[...]) * gs_ref[:, 0:1]


def _gffn(blk, xs, gs, kb, kbias, vb, vbias):
    spec = pltpu.PrefetchScalarGridSpec(
        num_scalar_prefetch=1,
        grid=(GR,),
        in_specs=[
            pl.BlockSpec((BT, D), lambda i, b: (i, 0)),
            pl.BlockSpec((BT, GW), lambda i, b: (i, 0)),
            pl.BlockSpec((D, H), lambda i, b: (0, b[i])),
            pl.BlockSpec((1, H), lambda i, b: (0, b[i])),
            pl.BlockSpec((H, D), lambda i, b: (0, b[i])),
            pl.BlockSpec((1, D), lambda i, b: (0, b[i])),
        ],
        out_specs=pl.BlockSpec((BT, D), lambda i, b: (i, 0)),
    )
    return pl.pallas_call(
        _gffn_body,
        grid_spec=spec,
        out_shape=jax.ShapeDtypeStruct((R, D), jnp.float32),
        compiler_params=pltpu.CompilerParams(
            dimension_semantics=("arbitrary",)),
    )(blk, xs, gs, kb, kbias, vb, vbias)


SHC = 1024
SNH = H // SHC


def _sffn_body(sk_ref, skb_ref, sv_ref, svb_ref, xb_ref, o_ref, skf, svf):
    hc = pl.program_id(0)
    skf[...] = sk_ref[:, 0, :].astype(jnp.bfloat16)
    svf[...] = sv_ref[:, 0, :].astype(jnp.bfloat16)

    def b_body(b, _):
        sl = pl.ds(b * BT, BT)
        h = jnp.dot(xb_ref[sl, :], skf[...], preferred_element_type=jnp.float32)
        h = jax.nn.gelu(h + skb_ref[0:1, :])
        y = jnp.dot(h.astype(jnp.bfloat16), svf[...],
                    preferred_element_type=jnp.float32)
        prev = jnp.where(hc == 0, svb_ref[0:1, :], o_ref[sl, :])
        o_ref[sl, :] = prev + y
        return 0

    lax.fori_loop(0, T // BT, b_body, 0)


def _sffn(sk, skb, sv, svb, xb):
    return pl.pallas_call(
        _sffn_body,
        grid=(SNH,),
        in_specs=[
            pl.BlockSpec((D, 1, SHC), lambda i: (0, 0, i)),
            pl.BlockSpec((1, SHC), lambda i: (0, i)),
            pl.BlockSpec((SHC, 1, D), lambda i: (i, 0, 0)),
            pl.BlockSpec((1, D), lambda i: (0, 0)),
            pl.BlockSpec((T, D), lambda i: (0, 0)),
        ],
        out_specs=pl.BlockSpec((T, D), lambda i: (0, 0)),
        out_shape=jax.ShapeDtypeStruct((T, D), jnp.float32),
        scratch_shapes=[pltpu.VMEM((D, SHC), jnp.bfloat16),
                        pltpu.VMEM((SHC, D), jnp.bfloat16)],
        compiler_params=pltpu.CompilerParams(
            dimension_semantics=("arbitrary",)),
    )(sk, skb, sv, svb, xb)


def _combine(ys, sh, pos0w, pos1w):
    mesh = plsc.VectorSubcoreMesh(core_axis_name="c", subcore_axis_name="s",
                                  num_cores=NC, num_subcores=NS)

    @functools.partial(
        pl.kernel,
        out_type=jax.ShapeDtypeStruct((T, D), jnp.float32),
        mesh=mesh,
        scratch_types=[
            pltpu.VMEM((CH, D), jnp.float32),
            pltpu.VMEM((CH, D), jnp.float32),
            pltpu.VMEM((CH, D), jnp.float32),
            pltpu.VMEM((TPW,), jnp.int32),
            pltpu.VMEM((TPW,), jnp.int32),
            pltpu.SemaphoreType.DMA,
        ],
    )
    def k(ys_hbm, sh_hbm, p0_hbm, p1_hbm, out_hbm,
          shbuf, b0, b1, idx0, idx1, sem):
        wid = lax.axis_index("s") * NC + lax.axis_index("c")
        base = wid * TPW
        pltpu.sync_copy(p0_hbm.at[wid], idx0)
        pltpu.sync_copy(p1_hbm.at[wid], idx1)
        for j in range(TPW // CH):
            off = j * CH
            iv0 = idx0[pl.ds(off, CH)]
            iv1 = idx1[pl.ds(off, CH)]
            pltpu.sync_copy(sh_hbm.at[pl.ds(base + off, CH)], shbuf)
            pltpu.async_copy(ys_hbm.at[iv0], b0, sem).wait()
            pltpu.async_copy(ys_hbm.at[iv1], b1, sem).wait()

            def row_body(r, _):
                def col_body(c, _):
                    sl = pl.ds(c * 16, 16)
                    shbuf[r, sl] = shbuf[r, sl] + b0[r, sl] + b1[r, sl]
                    return 0
                lax.fori_loop(0, D // 16, col_body, 0)
                return 0

            lax.fori_loop(0, CH, row_body, 0)
            pltpu.sync_copy(shbuf, out_hbm.at[pl.ds(base + off, CH)])

    return k(ys, sh, pos0w, pos1w)


def kernel(x, gate_weight, gate_bias, keys, key_bias, values, value_bias,
           s_keys, s_key_bias, s_values, s_value_bias):
    x2d = x.reshape(T, D)
    gb2d = gate_bias.reshape(1, E)
    kb = keys.astype(jnp.bfloat16).reshape(D, E * H)
    vb = values.astype(jnp.bfloat16).reshape(H, E * D)
    kbias = key_bias.reshape(1, E * H)
    vbias = value_bias.reshape(1, E * D)
    skb = s_key_bias.reshape(1, H)
    svb = s_value_bias.reshape(1, D)

    g0s, g1s, e0, e1, r0, r1, cnt = _router(x2d, gate_weight, gb2d)
    pos0, pos1, blk = _meta(cnt, e0, e1, r0, r1)
    pos0w = pos0.reshape(NW, TPW)
    pos1w = pos1.reshape(NW, TPW)

    xs, gs = _dispatch(x2d, g0s, g1s, pos0w, pos1w)
    ys = _gffn(blk.reshape(GR), xs, gs, kb, kbias, vb, vbias)
    sh = _sffn(s_keys, skb, s_values, svb, _cast_bf16(x2d))
    out = _combine(ys, sh, pos0w, pos1w)
    return out.reshape(x.shape)


# speedup vs baseline: 2.5165x; 1.0701x over previous
import functools

import jax
import jax.numpy as jnp
from jax import lax
from jax.experimental import pallas as pl
from jax.experimental.pallas import tpu as pltpu
from jax.experimental.pallas import tpu_sc as plsc

D = 1024
H = 4096
E = 8
T = 2048
K = 2
BT = 128
GR = (T * K + E * (BT - 1) + BT - 1) // BT
R = GR * BT
TB = 256
HC = 128
NH = H // HC
NC, NS = 2, 16
NW = NC * NS
TPW = T // NW
CH = 16
GW = 128


NBLK = T // TB


def _route_body(x_ref, gw_ref, gb_ref,
                xb_ref, g0_ref, g1_ref, pos0_ref, pos1_ref, blk_ref,
                carry, se0, se1, sr0, sr1):
    i = pl.program_id(0)

    @pl.when(i == 0)
    def _():
        carry[...] = jnp.zeros_like(carry)

    ua = lax.broadcasted_iota(jnp.int32, (E, E), 0)
    ub = lax.broadcasted_iota(jnp.int32, (E, E), 1)
    triu = (ua <= ub).astype(jnp.float32)

    @pl.when(i < NBLK)
    def _():
        xv = x_ref[...]
        xb_ref[...] = xv.astype(jnp.bfloat16)
        logits = jnp.dot(xv, gw_ref[...],
                         preferred_element_type=jnp.float32) + gb_ref[...]
        m = jnp.max(logits, axis=1, keepdims=True)
        ex = jnp.exp(logits - m)
        p = ex / jnp.sum(ex, axis=1, keepdims=True)

        lane = lax.broadcasted_iota(jnp.int32, (TB, E), 1).astype(jnp.float32)

        def pick_first_max(q):
            mv = jnp.max(q, axis=1, keepdims=True)
            eq = (q == mv).astype(jnp.float32)
            csum = jnp.dot(eq, triu, preferred_element_type=jnp.float32)
            oh = eq * (csum == 1.0).astype(jnp.float32)
            idx = jnp.sum(oh * lane, axis=1, keepdims=True)
            return mv, oh, idx

        m0, oh0, i0 = pick_first_max(p)
        m1, oh1, i1 = pick_first_max(p - 2.0 * oh0)

        ones_gw = jnp.ones((1, GW), jnp.float32)
        g0_ref[...] = m0 * ones_gw
        g1_ref[...] = m1 * ones_gw

        ohs = oh0 + oh1
        ra = lax.broadcasted_iota(jnp.int32, (TB, TB), 0)
        rb = lax.broadcasted_iota(jnp.int32, (TB, TB), 1)
        stril = (ra > rb).astype(jnp.float32)
        prior = carry[...] + jnp.dot(stril, ohs,
                                     preferred_element_type=jnp.float32)
        r0 = jnp.sum(oh0 * prior, axis=1, keepdims=True)
        r1 = jnp.sum(oh1 * (prior + oh0), axis=1, keepdims=True)
        sl = pl.ds(i * TB, TB)
        se0[sl, :] = i0.astype(jnp.int32)
        se1[sl, :] = i1.astype(jnp.int32)
        sr0[sl, :] = r0.astype(jnp.int32)
        sr1[sl, :] = r1.astype(jnp.int32)
        carry[...] = carry[...] + jnp.sum(ohs, axis=0, keepdims=True)

    @pl.when(i == NBLK)
    def _():
        cnt = carry[...]
        pad = jnp.ceil(cnt / BT) * BT
        upper = jnp.dot(pad, triu, preferred_element_type=jnp.float32)
        off = upper - pad

        lane_t = lax.broadcasted_iota(jnp.int32, (T, E), 1)

        def to_pos(e_s, r_s):
            oh = (lane_t == e_s[...]).astype(jnp.float32)
            return (r_s[...] +
                    jnp.sum(oh * off, axis=1, keepdims=True).astype(jnp.int32))

        pos0_ref[...] = to_pos(se0, sr0)
        pos1_ref[...] = to_pos(se1, sr1)

        brow = (lax.broadcasted_iota(jnp.int32, (GR, E), 0)
                .astype(jnp.float32) * BT)
        ge = (brow >= upper).astype(jnp.float32)
        blk = jnp.minimum(jnp.sum(ge, axis=1, keepdims=True), float(E - 1))
        blk_ref[...] = blk.astype(jnp.int32)


def _route(x2d, gw, gb2d):
    ilast = NBLK - 1

    def tok(i):
        return (jnp.minimum(i, ilast), 0)

    return pl.pallas_call(
        _route_body,
        grid=(NBLK + 1,),
        in_specs=[
            pl.BlockSpec((TB, D), tok),
            pl.BlockSpec((D, E), lambda i: (0, 0)),
            pl.BlockSpec((1, E), lambda i: (0, 0)),
        ],
        out_specs=[
            pl.BlockSpec((TB, D), tok),
            pl.BlockSpec((TB, GW), tok),
            pl.BlockSpec((TB, GW), tok),
            pl.BlockSpec((T, 1), lambda i: (0, 0)),
            pl.BlockSpec((T, 1), lambda i: (0, 0)),
            pl.BlockSpec((GR, 1), lambda i: (0, 0)),
        ],
        out_shape=[
            jax.ShapeDtypeStruct((T, D), jnp.bfloat16),
            jax.ShapeDtypeStruct((T, GW), jnp.float32),
            jax.ShapeDtypeStruct((T, GW), jnp.float32),
            jax.ShapeDtypeStruct((T, 1), jnp.int32),
            jax.ShapeDtypeStruct((T, 1), jnp.int32),
            jax.ShapeDtypeStruct((GR, 1), jnp.int32),
        ],
        scratch_shapes=[pltpu.VMEM((1, E), jnp.float32),
                        pltpu.VMEM((T, 1), jnp.int32),
                        pltpu.VMEM((T, 1), jnp.int32),
                        pltpu.VMEM((T, 1), jnp.int32),
                        pltpu.VMEM((T, 1), jnp.int32)],
        compiler_params=pltpu.CompilerParams(
            dimension_semantics=("arbitrary",)),
    )(x2d, gw, gb2d)


def _dispatch(x2d, g0s, g1s, pos0w, pos1w):
    mesh = plsc.VectorSubcoreMesh(core_axis_name="c", subcore_axis_name="s",
                                  num_cores=NC, num_subcores=NS)

    @functools.partial(
        pl.kernel,
        out_type=[
            jax.ShapeDtypeStruct((R, D), jnp.float32),
            jax.ShapeDtypeStruct((R, GW), jnp.float32),
        ],
        mesh=mesh,
        scratch_types=[
            pltpu.VMEM((TPW, D), jnp.float32),
            pltpu.VMEM((TPW, GW), jnp.float32),
            pltpu.VMEM((TPW, GW), jnp.float32),
            pltpu.VMEM((TPW,), jnp.int32),
            pltpu.VMEM((TPW,), jnp.int32),
            pltpu.SemaphoreType.DMA,
            pltpu.SemaphoreType.DMA,
            pltpu.SemaphoreType.DMA,
            pltpu.SemaphoreType.DMA,
        ],
    )
    def k(x_hbm, g0_hbm, g1_hbm, p0_hbm, p1_hbm, xs_hbm, gs_hbm,
          xbuf, g0buf, g1buf, idx0, idx1, s1, s2, s3, s4):
        wid = lax.axis_index("s") * NC + lax.axis_index("c")
        base = wid * TPW
        pltpu.sync_copy(p0_hbm.at[wid], idx0)
        pltpu.sync_copy(p1_hbm.at[wid], idx1)
        pltpu.sync_copy(x_hbm.at[pl.ds(base, TPW)], xbuf)
        pltpu.sync_copy(g0_hbm.at[pl.ds(base, TPW)], g0buf)
        pltpu.sync_copy(g1_hbm.at[pl.ds(base, TPW)], g1buf)
        h1 = pltpu.async_copy(xbuf, xs_hbm.at[idx0], s1)
        h2 = pltpu.async_copy(xbuf, xs_hbm.at[idx1], s2)
        h3 = pltpu.async_copy(g0buf, gs_hbm.at[idx0], s3)
        h4 = pltpu.async_copy(g1buf, gs_hbm.at[idx1], s4)
        h1.wait()
        h2.wait()
        h3.wait()
        h4.wait()

    return k(x2d, g0s, g1s, pos0w, pos1w)


def _cast_bf16_body(x_ref, o_ref):
    o_ref[...] = x_ref[...].astype(jnp.bfloat16)


def _cast_bf16(x2d):
    n, d = x2d.shape
    return pl.pallas_call(
        _cast_bf16_body,
        grid=(n // 512,),
        in_specs=[pl.BlockSpec((512, d), lambda i: (i, 0))],
        out_specs=pl.BlockSpec((512, d), lambda i: (i, 0)),
        out_shape=jax.ShapeDtypeStruct((n, d), jnp.bfloat16),
    )(x2d)


def _gffn_body(blk_ref, xs_ref, gs_ref, kb_ref, kbias_ref, vb_ref, vbias_ref,
               ys_ref):
    xb = xs_ref[...].astype(jnp.bfloat16)
    h = jnp.dot(xb, kb_ref[...], preferred_element_type=jnp.float32)
    h = jax.nn.gelu(h + kbias_ref[...])
    y = jnp.dot(h.astype(jnp.bfloat16), vb_ref[...],
                preferred_element_type=jnp.float32)
    ys_ref[...] = (y + vbias_ref[...]) * gs_ref[:, 0:1]


def _gffn(blk, xs, gs, kb, kbias, vb, vbias):
    spec = pltpu.PrefetchScalarGridSpec(
        num_scalar_prefetch=1,
        grid=(GR,),
        in_specs=[
            pl.BlockSpec((BT, D), lambda i, b: (i, 0)),
            pl.BlockSpec((BT, GW), lambda i, b: (i, 0)),
            pl.BlockSpec((D, H), lambda i, b: (0, b[i])),
            pl.BlockSpec((1, H), lambda i, b: (0, b[i])),
            pl.BlockSpec((H, D), lambda i, b: (0, b[i])),
            pl.BlockSpec((1, D), lambda i, b: (0, b[i])),
        ],
        out_specs=pl.BlockSpec((BT, D), lambda i, b: (i, 0)),
    )
    return pl.pallas_call(
        _gffn_body,
        grid_spec=spec,
        out_shape=jax.ShapeDtypeStruct((R, D), jnp.float32),
        compiler_params=pltpu.CompilerParams(
            dimension_semantics=("arbitrary",)),
    )(blk, xs, gs, kb, kbias, vb, vbias)


def _sffn_body(xb_ref, sk_ref, skb_ref, sv_ref, svb_ref, o_ref):
    h = jnp.dot(xb_ref[...], sk_ref[...], preferred_element_type=jnp.float32)
    h = jax.nn.gelu(h + skb_ref[...])
    y = jnp.dot(h.astype(jnp.bfloat16), sv_ref[...],
                preferred_element_type=jnp.float32)
    o_ref[...] = y + svb_ref[...]


def _sffn(xb, sk, skb, sv, svb):
    return pl.pallas_call(
        _sffn_body,
        grid=(T // BT,),
        in_specs=[
            pl.BlockSpec((BT, D), lambda i: (i, 0)),
            pl.BlockSpec((D, H), lambda i: (0, 0)),
            pl.BlockSpec((1, H), lambda i: (0, 0)),
            pl.BlockSpec((H, D), lambda i: (0, 0)),
            pl.BlockSpec((1, D), lambda i: (0, 0)),
        ],
        out_specs=pl.BlockSpec((BT, D), lambda i: (i, 0)),
        out_shape=jax.ShapeDtypeStruct((T, D), jnp.float32),
    )(xb, sk, skb, sv, svb)


def _combine(ys, sh, pos0w, pos1w):
    mesh = plsc.VectorSubcoreMesh(core_axis_name="c", subcore_axis_name="s",
                                  num_cores=NC, num_subcores=NS)

    @functools.partial(
        pl.kernel,
        out_type=jax.ShapeDtypeStruct((T, D), jnp.float32),
        mesh=mesh,
        scratch_types=[
            pltpu.VMEM((CH, D), jnp.float32),
            pltpu.VMEM((CH, D), jnp.float32),
            pltpu.VMEM((CH, D), jnp.float32),
            pltpu.VMEM((CH, D), jnp.float32),
            pltpu.VMEM((CH, D), jnp.float32),
            pltpu.VMEM((CH, D), jnp.float32),
            pltpu.VMEM((TPW,), jnp.int32),
            pltpu.VMEM((TPW,), jnp.int32),
            pltpu.SemaphoreType.DMA,
            pltpu.SemaphoreType.DMA,
            pltpu.SemaphoreType.DMA,
            pltpu.SemaphoreType.DMA,
            pltpu.SemaphoreType.DMA,
            pltpu.SemaphoreType.DMA,
            pltpu.SemaphoreType.DMA,
            pltpu.SemaphoreType.DMA,
        ],
    )
    def k(ys_hbm, sh_hbm, p0_hbm, p1_hbm, out_hbm,
          sh0, sh1, b0a, b0b, b1a, b1b, idx0, idx1,
          ss0, ss1, sb00, sb01, sb10, sb11, so0, so1):
        wid = lax.axis_index("s") * NC + lax.axis_index("c")
        base = wid * TPW
        pltpu.sync_copy(p0_hbm.at[wid], idx0)
        pltpu.sync_copy(p1_hbm.at[wid], idx1)
        shb = [sh0, sh1]
        b0b_ = [b0a, b0b]
        b1b_ = [b1a, b1b]
        ssem = [ss0, ss1]
        s0sem = [sb00, sb01]
        s1sem = [sb10, sb11]
        osem = [so0, so1]
        nchunk = TPW // CH
        out_pending = [None, None]

        def start(j):
            p = j % 2
            if out_pending[p] is not None:
                out_pending[p].wait()
                out_pending[p] = None
            off = j * CH
            iv0 = idx0[pl.ds(off, CH)]
            iv1 = idx1[pl.ds(off, CH)]
            hs = pltpu.async_copy(sh_hbm.at[pl.ds(base + off, CH)],
                                  shb[p], ssem[p])
            h0 = pltpu.async_copy(ys_hbm.at[iv0], b0b_[p], s0sem[p])
            h1 = pltpu.async_copy(ys_hbm.at[iv1], b1b_[p], s1sem[p])
            return (hs, h0, h1)

        cur = start(0)
        for j in range(nchunk):
            p = j % 2
            nxt = start(j + 1) if j + 1 < nchunk else None
            for h in cur:
                h.wait()
            sbuf, c0, c1 = shb[p], b0b_[p], b1b_[p]

            def row_body(r, _):
                def col_body(c, _):
                    sl = pl.ds(c * 16, 16)
                    sbuf[r, sl] = sbuf[r, sl] + c0[r, sl] + c1[r, sl]
                    return 0
                lax.fori_loop(0, D // 16, col_body, 0)
                return 0

            lax.fori_loop(0, CH, row_body, 0)
            out_pending[p] = pltpu.async_copy(
                sbuf, out_hbm.at[pl.ds(base + j * CH, CH)], osem[p])
            cur = nxt
        for p in range(2):
            if out_pending[p] is not None:
                out_pending[p].wait()

    return k(ys, sh, pos0w, pos1w)


def kernel(x, gate_weight, gate_bias, keys, key_bias, values, value_bias,
           s_keys, s_key_bias, s_values, s_value_bias):
    x2d = x.reshape(T, D)
    gb2d = gate_bias.reshape(1, E)
    kb = keys.astype(jnp.bfloat16).reshape(D, E * H)
    vb = values.astype(jnp.bfloat16).reshape(H, E * D)
    kbias = key_bias.reshape(1, E * H)
    vbias = value_bias.reshape(1, E * D)
    sk = s_keys.astype(jnp.bfloat16).reshape(D, H)
    sv = s_values.astype(jnp.bfloat16).reshape(H, D)
    skb = s_key_bias.reshape(1, H)
    svb = s_value_bias.reshape(1, D)

    xb, g0s, g1s, pos0, pos1, blk = _route(x2d, gate_weight, gb2d)
    pos0w = pos0.reshape(NW, TPW)
    pos1w = pos1.reshape(NW, TPW)

    xs, gs = _dispatch(x2d, g0s, g1s, pos0w, pos1w)
    ys = _gffn(blk.reshape(GR), xs, gs, kb, kbias, vb, vbias)
    sh = _sffn(xb, sk, skb, sv, svb)
    out = _combine(ys, sh, pos0w, pos1w)
    return out.reshape(x.shape)
